# baseline (speedup 1.0000x reference)
import functools

import jax
import jax.numpy as jnp
import numpy as np
from jax.experimental import pallas as pl
from jax.experimental.pallas import tpu as pltpu

_NP = 20
_S = 2048
_CB = 128
_T = 128
_NT = _S // _T


def _fspool_kernel(n_ref, x_ref, w_ref, pooled_ref, perm_ref, kref, iref):
    b = pl.program_id(0)
    n = n_ref[b]
    denom = jnp.maximum(n.astype(jnp.float32) - 1.0, 1.0)

    xt = x_ref[0].T

    sio_f = jax.lax.broadcasted_iota(jnp.int32, (_S, 1), 0).astype(jnp.float32)
    maskc = sio_f <= denom

    kref[...] = jnp.where(maskc, -xt, 99999.0 - xt)
    iref[...] = jax.lax.broadcasted_iota(jnp.int32, (_S, _CB), 0).astype(
        jnp.float32)

    rio = jax.lax.broadcasted_iota(jnp.int32, (_T, 1), 0)

    def _partner(x, j, rows):
        if j >= 8:
            x4 = x.reshape(rows // (2 * j), 2, j, _CB)
            x4 = jnp.concatenate([x4[:, 1:], x4[:, :1]], axis=1)
            return x4.reshape(rows, _CB)
        bitj0 = (rio & j) == 0
        return jnp.where(bitj0, pltpu.roll(x, rows - j, 0),
                         pltpu.roll(x, j, 0))

    def _cswap(key, idx, j, tmn):
        kp = _partner(key, j, key.shape[0])
        ip = _partner(idx, j, idx.shape[0])
        p_lt = (kp < key) | ((kp == key) & (ip < idx))
        take_p = p_lt != tmn
        return jnp.where(take_p, kp, key), jnp.where(take_p, ip, idx)

    def _fused_tile_pass(stages):
        def body(t, _):
            base = t * _T
            key = kref[pl.ds(base, _T), :]
            idx = iref[pl.ds(base, _T), :]
            for j, k in stages:
                bitj0 = (rio & j) == 0
                if k <= _T // 2:
                    tmn = ((rio & k) == 0) != bitj0
                else:
                    dirbit = (base & k) != 0
                    tmn = bitj0 == dirbit
                key, idx = _cswap(key, idx, j, tmn)
            kref[pl.ds(base, _T), :] = key
            iref[pl.ds(base, _T), :] = idx
            return 0

        jax.lax.fori_loop(0, _NT, body, 0)

    def _far_stage(j, k):
        jt = j // _T

        def body(p, _):
            low = p & (jt - 1)
            t0 = (p - low) * 2 + low
            b0 = t0 * _T
            b1 = b0 + j
            ka = kref[pl.ds(b0, _T), :]
            ia = iref[pl.ds(b0, _T), :]
            kb = kref[pl.ds(b1, _T), :]
            ib = iref[pl.ds(b1, _T), :]
            tmn = (b0 & k) != 0
            b_first = (kb < ka) | ((kb == ka) & (ib < ia))
            swap = b_first != tmn
            kref[pl.ds(b0, _T), :] = jnp.where(swap, kb, ka)
            iref[pl.ds(b0, _T), :] = jnp.where(swap, ib, ia)
            kref[pl.ds(b1, _T), :] = jnp.where(swap, ka, kb)
            iref[pl.ds(b1, _T), :] = jnp.where(swap, ia, ib)
            return 0

        jax.lax.fori_loop(0, _NT // 2, body, 0)

    first = []
    k = 2
    while k <= _T:
        j = k // 2
        while j >= 1:
            first.append((j, k))
            j //= 2
        k *= 2
    _fused_tile_pass(first)

    k = 2 * _T
    while k <= _S:
        j = k // 2
        while j >= _T:
            _far_stage(j, k)
            j //= 2
        _fused_tile_pass([(j2, k) for j2 in (64, 32, 16, 8, 4, 2, 1)
                          if j2 < _T])
        k *= 2

    idx = iref[...]
    perm_ref[0] = idx.astype(jnp.int32).T

    xs = -kref[...]
    sizes = jnp.minimum(sio_f / denom, 1.0)
    findex = float(_NP) * sizes
    fidx = jnp.floor(findex)
    frac = findex - fidx
    lane = jax.lax.broadcasted_iota(jnp.int32, (_S, _CB), 1).astype(jnp.float32)
    coef = jnp.where(lane == fidx, 1.0 - frac, 0.0)
    coef = coef + jnp.where(lane == jnp.minimum(fidx + 1.0, float(_NP)), frac, 0.0)
    coef = jnp.where(maskc, coef, 0.0)
    t = jax.lax.dot_general(xs, coef, (((0,), (0,)), ((), ())),
                            preferred_element_type=jnp.float32)
    pooled_ref[0, 0, 0] = jnp.sum(t * w_ref[...], axis=1)


@jax.jit
def kernel(x, weight, n):
    B, C, S = x.shape
    wpad = jnp.zeros((C, _CB), jnp.float32).at[:, : _NP + 1].set(weight)
    grid = (B, C // _CB)
    pooled, perm = pl.pallas_call(
        _fspool_kernel,
        grid=grid,
        in_specs=[
            pl.BlockSpec(memory_space=pltpu.SMEM),
            pl.BlockSpec((1, _CB, S), lambda b, c: (b, c, 0)),
            pl.BlockSpec((_CB, _CB), lambda b, c: (c, 0)),
        ],
        out_specs=[
            pl.BlockSpec((1, 1, 1, _CB), lambda b, c: (b, c, 0, 0)),
            pl.BlockSpec((1, _CB, S), lambda b, c: (b, c, 0)),
        ],
        out_shape=[
            jax.ShapeDtypeStruct((B, C // _CB, 1, _CB), jnp.float32),
            jax.ShapeDtypeStruct((B, C, S), jnp.int32),
        ],
        scratch_shapes=[
            pltpu.VMEM((_S, _CB), jnp.float32),
            pltpu.VMEM((_S, _CB), jnp.float32),
        ],
    )(n.astype(jnp.int32), x, wpad)
    return pooled.reshape(B, C), perm


# speedup vs baseline: 1.1724x; 1.1724x over previous
import functools

import jax
import jax.numpy as jnp
import numpy as np
from jax.experimental import pallas as pl
from jax.experimental.pallas import tpu as pltpu

_NP = 20
_S = 2048
_CB = 128
_T = 64
_NT = _S // _T


def _fspool_kernel(n_ref, x_ref, w_ref, pooled_ref, perm_ref, kref, iref):
    b = pl.program_id(0)
    n = n_ref[b]
    denom = jnp.maximum(n.astype(jnp.float32) - 1.0, 1.0)

    rio = jax.lax.broadcasted_iota(jnp.int32, (_T, 1), 0)
    rio_cb = jax.lax.broadcasted_iota(jnp.int32, (_T, _CB), 0)

    def _partner(x, j, rows):
        if j >= 8:
            x4 = x.reshape(rows // (2 * j), 2, j, _CB)
            x4 = jnp.concatenate([x4[:, 1:], x4[:, :1]], axis=1)
            return x4.reshape(rows, _CB)
        bitj0 = (rio & j) == 0
        return jnp.where(bitj0, pltpu.roll(x, rows - j, 0),
                         pltpu.roll(x, j, 0))

    def _cswap(key, idx, j, tmn):
        kp = _partner(key, j, key.shape[0])
        ip = _partner(idx, j, idx.shape[0])
        take_p = ((kp < key) != tmn) & (kp != key)
        return jnp.where(take_p, kp, key), jnp.where(take_p, ip, idx)

    def _stages_in_tile(key, idx, stages, base):
        for j, k in stages:
            bitj0 = (rio & j) == 0
            if k <= _T // 2:
                tmn = ((rio & k) == 0) != bitj0
            else:
                dirbit = (base & k) != 0
                tmn = bitj0 == dirbit
            key, idx = _cswap(key, idx, j, tmn)
        return key, idx

    first = []
    k = 2
    while k <= _T:
        j = k // 2
        while j >= 1:
            first.append((j, k))
            j //= 2
        k *= 2

    def body_first(t, _):
        xt2 = x_ref[0, :, pl.ds(t * 2 * _T, 2 * _T)].T
        for h in range(2):
            base = t * 2 * _T + h * _T
            xt = xt2[h * _T:(h + 1) * _T]
            idx = (rio_cb + base).astype(jnp.float32)
            valid = (rio + base).astype(jnp.float32) <= denom
            xm = xt - 99999.0
            key = jnp.where(
                valid, -xt, ((0.0 - xm) * 128.0 - 12795904.0) * 2048.0 + idx)
            key, idx = _stages_in_tile(key, idx, first, base)
            kref[pl.ds(base, _T), :] = key
            iref[pl.ds(base, _T), :] = idx
        return 0

    jax.lax.fori_loop(0, _NT // 2, body_first, 0)

    def _fused_tile_pass(stages):
        def body(t, _):
            base = t * _T
            key = kref[pl.ds(base, _T), :]
            idx = iref[pl.ds(base, _T), :]
            key, idx = _stages_in_tile(key, idx, stages, base)
            kref[pl.ds(base, _T), :] = key
            iref[pl.ds(base, _T), :] = idx
            return 0

        jax.lax.fori_loop(0, _NT, body, 0)

    def _far_stage(j, k):
        jt = j // _T

        def body(p, _):
            low = p & (jt - 1)
            t0 = (p - low) * 2 + low
            b0 = t0 * _T
            b1 = b0 + j
            ka = kref[pl.ds(b0, _T), :]
            ia = iref[pl.ds(b0, _T), :]
            kb = kref[pl.ds(b1, _T), :]
            ib = iref[pl.ds(b1, _T), :]
            tmn = (b0 & k) != 0
            swap = (kb < ka) != tmn
            kref[pl.ds(b0, _T), :] = jnp.where(swap, kb, ka)
            iref[pl.ds(b0, _T), :] = jnp.where(swap, ib, ia)
            kref[pl.ds(b1, _T), :] = jnp.where(swap, ka, kb)
            iref[pl.ds(b1, _T), :] = jnp.where(swap, ia, ib)
            return 0

        jax.lax.fori_loop(0, _NT // 2, body, 0)

    k = 2 * _T
    while k <= _S:
        j = k // 2
        while j >= _T:
            _far_stage(j, k)
            j //= 2
        _fused_tile_pass([(j2, k) for j2 in (32, 16, 8, 4, 2, 1)])
        k *= 2

    key = kref[...]
    idx = iref[...]

    sio = jax.lax.broadcasted_iota(jnp.int32, (_S, 1), 0)
    sio_f = sio.astype(jnp.float32)
    odd = (sio & 1) == 1
    kdn = pltpu.roll(key, _S - 1, 0)
    kup = pltpu.roll(key, 1, 0)
    idn = pltpu.roll(idx, _S - 1, 0)
    iup = pltpu.roll(idx, 1, 0)
    kp = jnp.where(odd, kup, kdn)
    ip = jnp.where(odd, iup, idn)
    fix = (kp == key) & ((ip < idx) != odd)
    idx = jnp.where(fix, ip, idx)
    idn = pltpu.roll(idx, _S - 1, 0)
    iup = pltpu.roll(idx, 1, 0)
    kp = jnp.where(odd, kdn, kup)
    ip = jnp.where(odd, idn, iup)
    inner = (sio > 0) & (sio < _S - 1)
    fix = (kp == key) & ((ip < idx) == odd) & inner
    idx = jnp.where(fix, ip, idx)

    perm_ref[0] = idx.astype(jnp.int32).T

    maskc = sio_f <= denom
    xs = -key
    sizes = jnp.minimum(sio_f / denom, 1.0)
    findex = float(_NP) * sizes
    fidx = jnp.floor(findex)
    frac = findex - fidx
    lane = jax.lax.broadcasted_iota(jnp.int32, (_S, _CB), 1).astype(jnp.float32)
    coef = jnp.where(lane == fidx, 1.0 - frac, 0.0)
    coef = coef + jnp.where(lane == jnp.minimum(fidx + 1.0, float(_NP)), frac, 0.0)
    coef = jnp.where(maskc, coef, 0.0)
    t = jax.lax.dot_general(xs, coef, (((0,), (0,)), ((), ())),
                            preferred_element_type=jnp.float32)
    pooled_ref[0, 0, 0] = jnp.sum(t * w_ref[...], axis=1)


@jax.jit
def kernel(x, weight, n):
    B, C, S = x.shape
    wpad = jnp.zeros((C, _CB), jnp.float32).at[:, : _NP + 1].set(weight)
    grid = (B, C // _CB)
    pooled, perm = pl.pallas_call(
        _fspool_kernel,
        grid=grid,
        in_specs=[
            pl.BlockSpec(memory_space=pltpu.SMEM),
            pl.BlockSpec((1, _CB, S), lambda b, c: (b, c, 0)),
            pl.BlockSpec((_CB, _CB), lambda b, c: (c, 0)),
        ],
        out_specs=[
            pl.BlockSpec((1, 1, 1, _CB), lambda b, c: (b, c, 0, 0)),
            pl.BlockSpec((1, _CB, S), lambda b, c: (b, c, 0)),
        ],
        out_shape=[
            jax.ShapeDtypeStruct((B, C // _CB, 1, _CB), jnp.float32),
            jax.ShapeDtypeStruct((B, C, S), jnp.int32),
        ],
        scratch_shapes=[
            pltpu.VMEM((_S, _CB), jnp.float32),
            pltpu.VMEM((_S, _CB), jnp.float32),
        ],
    )(n.astype(jnp.int32), x, wpad)
    return pooled.reshape(B, C), perm


# speedup vs baseline: 1.1862x; 1.0118x over previous
import functools

import jax
import jax.numpy as jnp
import numpy as np
from jax.experimental import pallas as pl
from jax.experimental.pallas import tpu as pltpu

_NP = 20
_S = 2048
_CB = 128
_T = 64
_NT = _S // _T


def _fspool_kernel(n_ref, x_ref, w_ref, pooled_ref, perm_ref, kref, iref):
    b = pl.program_id(0)
    n = n_ref[b]
    denom = jnp.maximum(n.astype(jnp.float32) - 1.0, 1.0)

    rio = jax.lax.broadcasted_iota(jnp.int32, (_T, 1), 0)
    rio_cb = jax.lax.broadcasted_iota(jnp.int32, (_T, _CB), 0)

    def _partner(x, j, rows):
        if j >= 8:
            x4 = x.reshape(rows // (2 * j), 2, j, _CB)
            x4 = jnp.concatenate([x4[:, 1:], x4[:, :1]], axis=1)
            return x4.reshape(rows, _CB)
        bitj0 = (rio & j) == 0
        return jnp.where(bitj0, pltpu.roll(x, rows - j, 0),
                         pltpu.roll(x, j, 0))

    def _cswap(key, idx, j, tmn):
        kp = _partner(key, j, key.shape[0])
        ip = _partner(idx, j, idx.shape[0])
        take_p = ((kp < key) != tmn) & (kp != key)
        return jnp.where(take_p, kp, key), jnp.where(take_p, ip, idx)

    def _stages_in_tile(key, idx, stages, base):
        for j, k in stages:
            bitj0 = (rio & j) == 0
            if k <= _T // 2:
                tmn = ((rio & k) == 0) != bitj0
            else:
                dirbit = (base & k) != 0
                tmn = bitj0 == dirbit
            key, idx = _cswap(key, idx, j, tmn)
        return key, idx

    first = []
    k = 2
    while k <= _T:
        j = k // 2
        while j >= 1:
            first.append((j, k))
            j //= 2
        k *= 2

    xt_all = x_ref[0].T
    sio_cb = jax.lax.broadcasted_iota(jnp.int32, (_S, _CB), 0)
    idx_all = sio_cb.astype(jnp.float32)
    sio1 = jax.lax.broadcasted_iota(jnp.int32, (_S, 1), 0)
    valid_all = sio1.astype(jnp.float32) <= denom
    xm_all = xt_all - 99999.0
    kref[...] = jnp.where(
        valid_all, -xt_all,
        ((0.0 - xm_all) * 128.0 - 12795904.0) * 2048.0 + idx_all)
    iref[...] = idx_all

    def body_first(t, _):
        base = t * _T
        key = kref[pl.ds(base, _T), :]
        idx = iref[pl.ds(base, _T), :]
        key, idx = _stages_in_tile(key, idx, first, base)
        kref[pl.ds(base, _T), :] = key
        iref[pl.ds(base, _T), :] = idx
        return 0

    jax.lax.fori_loop(0, _NT, body_first, 0)

    def _fused_tile_pass(stages):
        def body(t, _):
            base = t * _T
            key = kref[pl.ds(base, _T), :]
            idx = iref[pl.ds(base, _T), :]
            key, idx = _stages_in_tile(key, idx, stages, base)
            kref[pl.ds(base, _T), :] = key
            iref[pl.ds(base, _T), :] = idx
            return 0

        jax.lax.fori_loop(0, _NT, body, 0)

    def _far_stage(j, k):
        jt = j // _T

        def body(p, _):
            low = p & (jt - 1)
            t0 = (p - low) * 2 + low
            b0 = t0 * _T
            b1 = b0 + j
            ka = kref[pl.ds(b0, _T), :]
            ia = iref[pl.ds(b0, _T), :]
            kb = kref[pl.ds(b1, _T), :]
            ib = iref[pl.ds(b1, _T), :]
            tmn = (b0 & k) != 0
            swap = (kb < ka) != tmn
            kref[pl.ds(b0, _T), :] = jnp.where(swap, kb, ka)
            iref[pl.ds(b0, _T), :] = jnp.where(swap, ib, ia)
            kref[pl.ds(b1, _T), :] = jnp.where(swap, ka, kb)
            iref[pl.ds(b1, _T), :] = jnp.where(swap, ia, ib)
            return 0

        jax.lax.fori_loop(0, _NT // 2, body, 0)

    k = 2 * _T
    while k <= _S:
        j = k // 2
        while j >= _T:
            _far_stage(j, k)
            j //= 2
        _fused_tile_pass([(j2, k) for j2 in (32, 16, 8, 4, 2, 1)])
        k *= 2

    key = kref[...]
    idx = iref[...]

    sio = jax.lax.broadcasted_iota(jnp.int32, (_S, 1), 0)
    sio_f = sio.astype(jnp.float32)
    odd = (sio & 1) == 1
    kdn = pltpu.roll(key, _S - 1, 0)
    kup = pltpu.roll(key, 1, 0)
    idn = pltpu.roll(idx, _S - 1, 0)
    iup = pltpu.roll(idx, 1, 0)
    kp = jnp.where(odd, kup, kdn)
    ip = jnp.where(odd, iup, idn)
    fix = (kp == key) & ((ip < idx) != odd)
    idx = jnp.where(fix, ip, idx)
    idn = pltpu.roll(idx, _S - 1, 0)
    iup = pltpu.roll(idx, 1, 0)
    kp = jnp.where(odd, kdn, kup)
    ip = jnp.where(odd, idn, iup)
    inner = (sio > 0) & (sio < _S - 1)
    fix = (kp == key) & ((ip < idx) == odd) & inner
    idx = jnp.where(fix, ip, idx)

    perm_ref[0] = idx.astype(jnp.int32).T

    maskc = sio_f <= denom
    xs = -key
    sizes = jnp.minimum(sio_f / denom, 1.0)
    findex = float(_NP) * sizes
    fidx = jnp.floor(findex)
    frac = findex - fidx
    lane = jax.lax.broadcasted_iota(jnp.int32, (_S, _CB), 1).astype(jnp.float32)
    coef = jnp.where(lane == fidx, 1.0 - frac, 0.0)
    coef = coef + jnp.where(lane == jnp.minimum(fidx + 1.0, float(_NP)), frac, 0.0)
    coef = jnp.where(maskc, coef, 0.0)
    t = jax.lax.dot_general(xs, coef, (((0,), (0,)), ((), ())),
                            preferred_element_type=jnp.float32)
    pooled_ref[0, 0, 0] = jnp.sum(t * w_ref[...], axis=1)


@jax.jit
def kernel(x, weight, n):
    B, C, S = x.shape
    wpad = jnp.zeros((C, _CB), jnp.float32).at[:, : _NP + 1].set(weight)
    grid = (B, C // _CB)
    pooled, perm = pl.pallas_call(
        _fspool_kernel,
        grid=grid,
        in_specs=[
            pl.BlockSpec(memory_space=pltpu.SMEM),
            pl.BlockSpec((1, _CB, S), lambda b, c: (b, c, 0)),
            pl.BlockSpec((_CB, _CB), lambda b, c: (c, 0)),
        ],
        out_specs=[
            pl.BlockSpec((1, 1, 1, _CB), lambda b, c: (b, c, 0, 0)),
            pl.BlockSpec((1, _CB, S), lambda b, c: (b, c, 0)),
        ],
        out_shape=[
            jax.ShapeDtypeStruct((B, C // _CB, 1, _CB), jnp.float32),
            jax.ShapeDtypeStruct((B, C, S), jnp.int32),
        ],
        scratch_shapes=[
            pltpu.VMEM((_S, _CB), jnp.float32),
            pltpu.VMEM((_S, _CB), jnp.float32),
        ],
    )(n.astype(jnp.int32), x, wpad)
    return pooled.reshape(B, C), perm


# speedup vs baseline: 1.2384x; 1.0440x over previous
import functools

import jax
import jax.numpy as jnp
import numpy as np
from jax.experimental import pallas as pl
from jax.experimental.pallas import tpu as pltpu

_NP = 20
_S = 2048
_CB = 128
_T = 64
_NT = _S // _T


def _fspool_kernel(n_ref, x_ref, w_ref, pooled_ref, perm_ref, kref, iref):
    b = pl.program_id(0)
    n = n_ref[b]
    denom = jnp.maximum(n.astype(jnp.float32) - 1.0, 1.0)

    rio = jax.lax.broadcasted_iota(jnp.int32, (_T, 1), 0)
    rio_cb = jax.lax.broadcasted_iota(jnp.int32, (_T, _CB), 0)

    def _partner(x, j, rows):
        if j >= 8:
            x4 = x.reshape(rows // (2 * j), 2, j, _CB)
            x4 = jnp.concatenate([x4[:, 1:], x4[:, :1]], axis=1)
            return x4.reshape(rows, _CB)
        bitj0 = (rio & j) == 0
        return jnp.where(bitj0, pltpu.roll(x, rows - j, 0),
                         pltpu.roll(x, j, 0))

    def _cswap(key, idx, j, tmn):
        kp = _partner(key, j, key.shape[0])
        ip = _partner(idx, j, idx.shape[0])
        take_p = ((kp < key) != tmn) & (kp != key)
        return jnp.where(take_p, kp, key), jnp.where(take_p, ip, idx)

    def _stages_in_tile(key, idx, stages, base):
        for j, k in stages:
            bitj0 = (rio & j) == 0
            if k <= _T // 2:
                tmn = ((rio & k) == 0) != bitj0
            else:
                dirbit = (base & k) != 0
                tmn = bitj0 == dirbit
            key, idx = _cswap(key, idx, j, tmn)
        return key, idx

    first = []
    k = 2
    while k <= _T:
        j = k // 2
        while j >= 1:
            first.append((j, k))
            j //= 2
        k *= 2

    xt_all = x_ref[0].T
    sio_cb = jax.lax.broadcasted_iota(jnp.int32, (_S, _CB), 0)
    idx_all = sio_cb.astype(jnp.float32)
    sio1 = jax.lax.broadcasted_iota(jnp.int32, (_S, 1), 0)
    valid_all = sio1.astype(jnp.float32) <= denom
    xm_all = xt_all - 99999.0
    kref[...] = jnp.where(
        valid_all, -xt_all,
        ((0.0 - xm_all) * 128.0 - 12795904.0) * 2048.0 + idx_all)
    iref[...] = idx_all

    def body_first(t, _):
        base = t * _T
        key = kref[pl.ds(base, _T), :]
        idx = iref[pl.ds(base, _T), :]
        key, idx = _stages_in_tile(key, idx, first, base)
        kref[pl.ds(base, _T), :] = key
        iref[pl.ds(base, _T), :] = idx
        return 0

    jax.lax.fori_loop(0, _NT, body_first, 0)

    def _fused_tile_pass(stages):
        def body(t, _):
            base = t * _T
            key = kref[pl.ds(base, _T), :]
            idx = iref[pl.ds(base, _T), :]
            key, idx = _stages_in_tile(key, idx, stages, base)
            kref[pl.ds(base, _T), :] = key
            iref[pl.ds(base, _T), :] = idx
            return 0

        jax.lax.fori_loop(0, _NT, body, 0)

    def _far_stage(j, k):
        jt = j // _T

        def body(p, _):
            low = p & (jt - 1)
            t0 = (p - low) * 2 + low
            b0 = t0 * _T
            b1 = b0 + j
            ka = kref[pl.ds(b0, _T), :]
            ia = iref[pl.ds(b0, _T), :]
            kb = kref[pl.ds(b1, _T), :]
            ib = iref[pl.ds(b1, _T), :]
            tmn = (b0 & k) != 0
            swap = (kb < ka) != tmn
            kref[pl.ds(b0, _T), :] = jnp.where(swap, kb, ka)
            iref[pl.ds(b0, _T), :] = jnp.where(swap, ib, ia)
            kref[pl.ds(b1, _T), :] = jnp.where(swap, ka, kb)
            iref[pl.ds(b1, _T), :] = jnp.where(swap, ia, ib)
            return 0

        jax.lax.fori_loop(0, _NT // 2, body, 0)

    def _merged_tail_pass(k):
        tails = [(j2, k) for j2 in (32, 16, 8, 4, 2, 1)]

        def body(p, _):
            b0 = 2 * p * _T
            b1 = b0 + _T
            ka = kref[pl.ds(b0, _T), :]
            ia = iref[pl.ds(b0, _T), :]
            kb = kref[pl.ds(b1, _T), :]
            ib = iref[pl.ds(b1, _T), :]
            tmn = (b0 & k) != 0
            swap = (kb < ka) != tmn
            ka, kb = jnp.where(swap, kb, ka), jnp.where(swap, ka, kb)
            ia, ib = jnp.where(swap, ib, ia), jnp.where(swap, ia, ib)
            ka, ia = _stages_in_tile(ka, ia, tails, b0)
            kb, ib = _stages_in_tile(kb, ib, tails, b1)
            kref[pl.ds(b0, _T), :] = ka
            iref[pl.ds(b0, _T), :] = ia
            kref[pl.ds(b1, _T), :] = kb
            iref[pl.ds(b1, _T), :] = ib
            return 0

        jax.lax.fori_loop(0, _NT // 2, body, 0)

    k = 2 * _T
    while k <= _S:
        j = k // 2
        while j >= 2 * _T:
            _far_stage(j, k)
            j //= 2
        _merged_tail_pass(k)
        k *= 2

    key = kref[...]
    idx = iref[...]

    sio = jax.lax.broadcasted_iota(jnp.int32, (_S, 1), 0)
    sio_f = sio.astype(jnp.float32)
    odd = (sio & 1) == 1
    kdn = pltpu.roll(key, _S - 1, 0)
    kup = pltpu.roll(key, 1, 0)
    idn = pltpu.roll(idx, _S - 1, 0)
    iup = pltpu.roll(idx, 1, 0)
    kp = jnp.where(odd, kup, kdn)
    ip = jnp.where(odd, iup, idn)
    fix = (kp == key) & ((ip < idx) != odd)
    idx = jnp.where(fix, ip, idx)
    idn = pltpu.roll(idx, _S - 1, 0)
    iup = pltpu.roll(idx, 1, 0)
    kp = jnp.where(odd, kdn, kup)
    ip = jnp.where(odd, idn, iup)
    inner = (sio > 0) & (sio < _S - 1)
    fix = (kp == key) & ((ip < idx) == odd) & inner
    idx = jnp.where(fix, ip, idx)

    perm_ref[0] = idx.astype(jnp.int32).T

    maskc = sio_f <= denom
    xs = -key
    sizes = jnp.minimum(sio_f / denom, 1.0)
    findex = float(_NP) * sizes
    fidx = jnp.floor(findex)
    frac = findex - fidx
    lane = jax.lax.broadcasted_iota(jnp.int32, (_S, _CB), 1).astype(jnp.float32)
    coef = jnp.where(lane == fidx, 1.0 - frac, 0.0)
    coef = coef + jnp.where(lane == jnp.minimum(fidx + 1.0, float(_NP)), frac, 0.0)
    coef = jnp.where(maskc, coef, 0.0)
    t = jax.lax.dot_general(xs, coef, (((0,), (0,)), ((), ())),
                            preferred_element_type=jnp.float32)
    pooled_ref[0, 0, 0] = jnp.sum(t * w_ref[...], axis=1)


@jax.jit
def kernel(x, weight, n):
    B, C, S = x.shape
    wpad = jnp.zeros((C, _CB), jnp.float32).at[:, : _NP + 1].set(weight)
    grid = (B, C // _CB)
    pooled, perm = pl.pallas_call(
        _fspool_kernel,
        grid=grid,
        in_specs=[
            pl.BlockSpec(memory_space=pltpu.SMEM),
            pl.BlockSpec((1, _CB, S), lambda b, c: (b, c, 0)),
            pl.BlockSpec((_CB, _CB), lambda b, c: (c, 0)),
        ],
        out_specs=[
            pl.BlockSpec((1, 1, 1, _CB), lambda b, c: (b, c, 0, 0)),
            pl.BlockSpec((1, _CB, S), lambda b, c: (b, c, 0)),
        ],
        out_shape=[
            jax.ShapeDtypeStruct((B, C // _CB, 1, _CB), jnp.float32),
            jax.ShapeDtypeStruct((B, C, S), jnp.int32),
        ],
        scratch_shapes=[
            pltpu.VMEM((_S, _CB), jnp.float32),
            pltpu.VMEM((_S, _CB), jnp.float32),
        ],
    )(n.astype(jnp.int32), x, wpad)
    return pooled.reshape(B, C), perm


# speedup vs baseline: 1.2385x; 1.0001x over previous
import jax
import jax.numpy as jnp
from jax.experimental import pallas as pl
from jax.experimental.pallas import tpu as pltpu

_NP = 20
_S = 2048
_CB = 128
_T = 64
_NT = _S // _T


def _fspool_kernel(n_ref, x_ref, w_ref, pooled_ref, perm_ref, kref, iref):
    b = pl.program_id(0)
    n = n_ref[b]
    denom = jnp.maximum(n.astype(jnp.float32) - 1.0, 1.0)

    rio = jax.lax.broadcasted_iota(jnp.int32, (_T, 1), 0)

    def _partner(x, j, rows):
        if j >= 8:
            x4 = x.reshape(rows // (2 * j), 2, j, _CB)
            x4 = jnp.concatenate([x4[:, 1:], x4[:, :1]], axis=1)
            return x4.reshape(rows, _CB)
        bitj0 = (rio & j) == 0
        return jnp.where(bitj0, pltpu.roll(x, rows - j, 0),
                         pltpu.roll(x, j, 0))

    def _cswap(key, idx, j, tmn):
        kp = _partner(key, j, key.shape[0])
        ip = _partner(idx, j, idx.shape[0])
        take_p = ((kp < key) != tmn) & (kp != key)
        return jnp.where(take_p, kp, key), jnp.where(take_p, ip, idx)

    def _stages_in_tile(key, idx, stages, base):
        for j, k in stages:
            bitj0 = (rio & j) == 0
            if k <= _T // 2:
                tmn = ((rio & k) == 0) != bitj0
            else:
                dirbit = (base & k) != 0
                tmn = bitj0 == dirbit
            key, idx = _cswap(key, idx, j, tmn)
        return key, idx

    first = []
    k = 2
    while k <= _T:
        j = k // 2
        while j >= 1:
            first.append((j, k))
            j //= 2
        k *= 2

    xt_all = x_ref[0].T
    sio_cb = jax.lax.broadcasted_iota(jnp.int32, (_S, _CB), 0)
    idx_all = sio_cb.astype(jnp.float32)
    sio1 = jax.lax.broadcasted_iota(jnp.int32, (_S, 1), 0)
    valid_all = (sio1.astype(jnp.float32) / denom) <= 1.0
    xm_all = xt_all - 99999.0
    kref[...] = jnp.where(
        valid_all, -xt_all,
        ((0.0 - xm_all) * 128.0 - 12795904.0) * 2048.0 + idx_all)
    iref[...] = idx_all

    def body_first(t, _):
        base = t * _T
        key = kref[pl.ds(base, _T), :]
        idx = iref[pl.ds(base, _T), :]
        key, idx = _stages_in_tile(key, idx, first, base)
        kref[pl.ds(base, _T), :] = key
        iref[pl.ds(base, _T), :] = idx
        return 0

    jax.lax.fori_loop(0, _NT, body_first, 0)

    def _fused_tile_pass(stages):
        def body(t, _):
            base = t * _T
            key = kref[pl.ds(base, _T), :]
            idx = iref[pl.ds(base, _T), :]
            key, idx = _stages_in_tile(key, idx, stages, base)
            kref[pl.ds(base, _T), :] = key
            iref[pl.ds(base, _T), :] = idx
            return 0

        jax.lax.fori_loop(0, _NT, body, 0)

    def _far_stage(j, k):
        jt = j // _T

        def body(p, _):
            low = p & (jt - 1)
            t0 = (p - low) * 2 + low
            b0 = t0 * _T
            b1 = b0 + j
            ka = kref[pl.ds(b0, _T), :]
            ia = iref[pl.ds(b0, _T), :]
            kb = kref[pl.ds(b1, _T), :]
            ib = iref[pl.ds(b1, _T), :]
            tmn = (b0 & k) != 0
            swap = (kb < ka) != tmn
            kref[pl.ds(b0, _T), :] = jnp.where(swap, kb, ka)
            iref[pl.ds(b0, _T), :] = jnp.where(swap, ib, ia)
            kref[pl.ds(b1, _T), :] = jnp.where(swap, ka, kb)
            iref[pl.ds(b1, _T), :] = jnp.where(swap, ia, ib)
            return 0

        jax.lax.fori_loop(0, _NT // 2, body, 0)

    def _merged_tail_pass(k):
        tails = [(j2, k) for j2 in (32, 16, 8, 4, 2, 1)]

        def body(p, _):
            b0 = 2 * p * _T
            b1 = b0 + _T
            ka = kref[pl.ds(b0, _T), :]
            ia = iref[pl.ds(b0, _T), :]
            kb = kref[pl.ds(b1, _T), :]
            ib = iref[pl.ds(b1, _T), :]
            tmn = (b0 & k) != 0
            swap = (kb < ka) != tmn
            ka, kb = jnp.where(swap, kb, ka), jnp.where(swap, ka, kb)
            ia, ib = jnp.where(swap, ib, ia), jnp.where(swap, ia, ib)
            ka, ia = _stages_in_tile(ka, ia, tails, b0)
            kb, ib = _stages_in_tile(kb, ib, tails, b1)
            kref[pl.ds(b0, _T), :] = ka
            iref[pl.ds(b0, _T), :] = ia
            kref[pl.ds(b1, _T), :] = kb
            iref[pl.ds(b1, _T), :] = ib
            return 0

        jax.lax.fori_loop(0, _NT // 2, body, 0)

    k = 2 * _T
    while k <= _S:
        j = k // 2
        while j >= 2 * _T:
            _far_stage(j, k)
            j //= 2
        _merged_tail_pass(k)
        k *= 2

    key = kref[...]
    idx = iref[...]

    sio = jax.lax.broadcasted_iota(jnp.int32, (_S, 1), 0)
    sio_f = sio.astype(jnp.float32)
    odd = (sio & 1) == 1
    kdn = pltpu.roll(key, _S - 1, 0)
    kup = pltpu.roll(key, 1, 0)
    idn = pltpu.roll(idx, _S - 1, 0)
    iup = pltpu.roll(idx, 1, 0)
    kp = jnp.where(odd, kup, kdn)
    ip = jnp.where(odd, iup, idn)
    fix = (kp == key) & ((ip < idx) != odd)
    idx = jnp.where(fix, ip, idx)
    idn = pltpu.roll(idx, _S - 1, 0)
    iup = pltpu.roll(idx, 1, 0)
    kp = jnp.where(odd, kdn, kup)
    ip = jnp.where(odd, idn, iup)
    inner = (sio > 0) & (sio < _S - 1)
    fix = (kp == key) & ((ip < idx) == odd) & inner
    idx = jnp.where(fix, ip, idx)

    perm_ref[0] = idx.astype(jnp.int32).T

    maskc = (sio_f / denom) <= 1.0
    xs = -key
    sizes = jnp.minimum(sio_f / denom, 1.0)
    findex = float(_NP) * sizes
    fidx = jnp.floor(findex)
    frac = findex - fidx
    lane = jax.lax.broadcasted_iota(jnp.int32, (_S, _CB), 1).astype(jnp.float32)
    coef = jnp.where(lane == fidx, 1.0 - frac, 0.0)
    coef = coef + jnp.where(lane == jnp.minimum(fidx + 1.0, float(_NP)), frac, 0.0)
    coef = jnp.where(maskc, coef, 0.0)
    t = jax.lax.dot_general(xs, coef, (((0,), (0,)), ((), ())),
                            preferred_element_type=jnp.float32)
    pooled_ref[0, 0, 0] = jnp.sum(t * w_ref[...], axis=1)


@jax.jit
def kernel(x, weight, n):
    B, C, S = x.shape
    wpad = jnp.zeros((C, _CB), jnp.float32).at[:, : _NP + 1].set(weight)
    grid = (B, C // _CB)
    pooled, perm = pl.pallas_call(
        _fspool_kernel,
        grid=grid,
        in_specs=[
            pl.BlockSpec(memory_space=pltpu.SMEM),
            pl.BlockSpec((1, _CB, S), lambda b, c: (b, c, 0)),
            pl.BlockSpec((_CB, _CB), lambda b, c: (c, 0)),
        ],
        out_specs=[
            pl.BlockSpec((1, 1, 1, _CB), lambda b, c: (b, c, 0, 0)),
            pl.BlockSpec((1, _CB, S), lambda b, c: (b, c, 0)),
        ],
        out_shape=[
            jax.ShapeDtypeStruct((B, C // _CB, 1, _CB), jnp.float32),
            jax.ShapeDtypeStruct((B, C, S), jnp.int32),
        ],
        scratch_shapes=[
            pltpu.VMEM((_S, _CB), jnp.float32),
            pltpu.VMEM((_S, _CB), jnp.float32),
        ],
    )(n.astype(jnp.int32), x, wpad)
    return pooled.reshape(B, C), perm
